# Initial kernel scaffold; baseline (speedup 1.0000x reference)
import jax
import jax.numpy as jnp
from jax.experimental import pallas as pl


def kernel(x, edge_index, W_in, b_in, W_edge, b_edge, W_out, b_out):
    raise NotImplementedError("write your pallas kernel here")


# speedup vs baseline: 30.8189x; 30.8189x over previous
import functools

import jax
import jax.numpy as jnp
from jax import lax
from jax.experimental import pallas as pl
from jax.experimental.pallas import tpu as pltpu
from jax.experimental.pallas import tpu_sc as plsc

_NC = 2
_NS = 16
_NW = _NC * _NS
_L = 16


def _feats_body(x_ref, w_in_ref, wcat_ref, c_ref, o_ref):
    m8 = lax.dot_general(w_in_ref[...], wcat_ref[...],
                         (((1,), (0,)), ((), ())),
                         precision=lax.Precision.HIGHEST,
                         preferred_element_type=jnp.float32)
    o_ref[...] = lax.dot_general(m8, x_ref[...],
                                 (((0,), (1,)), ((), ())),
                                 precision=lax.Precision.HIGHEST,
                                 preferred_element_type=jnp.float32) + c_ref[...]


def _reduce_body(p_ref, b_ref, o_ref):
    o_ref[...] = jnp.sum(p_ref[...], axis=0, keepdims=True) + b_ref[...]


def _make_sc_edge_kernel(n_nodes, n_edges):
    epw = n_edges // _NW
    acc_len = 2 * n_nodes
    mesh = plsc.VectorSubcoreMesh(core_axis_name="c", subcore_axis_name="s",
                                  num_cores=_NC, num_subcores=_NS)

    @functools.partial(
        pl.kernel,
        out_type=jax.ShapeDtypeStruct((_NW, acc_len), jnp.float32),
        mesh=mesh,
        compiler_params=pltpu.CompilerParams(needs_layout_passes=False),
        scratch_types=[
            pltpu.VMEM((epw,), jnp.int32),
            pltpu.VMEM((epw,), jnp.int32),
            pltpu.VMEM((n_nodes,), jnp.float32),
            pltpu.VMEM((n_nodes,), jnp.float32),
            pltpu.VMEM((n_nodes,), jnp.float32),
            pltpu.VMEM((n_nodes,), jnp.float32),
            pltpu.VMEM((acc_len,), jnp.float32),
        ],
    )
    def sc_edge_kernel(src_hbm, dst_hbm, feats_hbm, part_hbm,
                       srcv, dstv, g0v, g1v, pv, qv, accv):
        wid = lax.axis_index("s") * _NC + lax.axis_index("c")
        base = wid * epw
        pltpu.sync_copy(src_hbm.at[pl.ds(base, epw)], srcv)
        pltpu.sync_copy(dst_hbm.at[pl.ds(base, epw)], dstv)
        pltpu.sync_copy(feats_hbm.at[0], g0v)
        pltpu.sync_copy(feats_hbm.at[1], g1v)
        pltpu.sync_copy(feats_hbm.at[2], pv)
        pltpu.sync_copy(feats_hbm.at[3], qv)

        def zero_body(i, carry):
            accv[pl.ds(i * _L, _L)] = jnp.zeros((_L,), jnp.float32)
            return carry
        lax.fori_loop(0, acc_len // _L, zero_body, 0)

        def edge_body(i, carry):
            s = srcv[pl.ds(i * _L, _L)]
            d = dstv[pl.ds(i * _L, _L)]
            pp = plsc.load_gather(pv, [s])
            qq = plsc.load_gather(qv, [d])
            w = 1.0 / (1.0 + jnp.exp(-(pp + qq)))
            w0 = w * plsc.load_gather(g0v, [s])
            w1 = w * plsc.load_gather(g1v, [s])
            d2 = d + d
            plsc.addupdate_scatter(accv, [d2], w0)
            plsc.addupdate_scatter(accv, [d2 + 1], w1)
            return carry
        lax.fori_loop(0, epw // _L, edge_body, 0)

        pltpu.sync_copy(accv, part_hbm.at[wid])

    return sc_edge_kernel


def kernel(x, edge_index, W_in, b_in, W_edge, b_edge, W_out, b_out):
    n_nodes, in_feats = x.shape
    n_edges = edge_index.shape[1]
    hidden = W_in.shape[1]

    wcat = jnp.concatenate(
        [W_out, W_edge[:hidden], W_edge[hidden:],
         jnp.zeros((hidden, 4), jnp.float32)], axis=1)
    ext = jnp.zeros((8,), jnp.float32).at[3].set(b_edge[0])
    c_col = (b_in @ wcat + ext).reshape(8, 1)

    feats = pl.pallas_call(
        _feats_body,
        out_shape=jax.ShapeDtypeStruct((8, n_nodes), jnp.float32),
        in_specs=[
            pl.BlockSpec((n_nodes, in_feats), lambda: (0, 0)),
            pl.BlockSpec((in_feats, hidden), lambda: (0, 0)),
            pl.BlockSpec((hidden, 8), lambda: (0, 0)),
            pl.BlockSpec((8, 1), lambda: (0, 0)),
        ],
        out_specs=pl.BlockSpec((8, n_nodes), lambda: (0, 0)),
    )(x, W_in, wcat, c_col)

    src = edge_index[0]
    dst = edge_index[1]
    partials = _make_sc_edge_kernel(n_nodes, n_edges)(src, dst, feats)

    acc_len = 2 * n_nodes
    btile = jnp.tile(b_out, acc_len // 2).reshape(1, acc_len)
    out_flat = pl.pallas_call(
        _reduce_body,
        out_shape=jax.ShapeDtypeStruct((1, acc_len), jnp.float32),
        in_specs=[
            pl.BlockSpec((_NW, acc_len), lambda: (0, 0)),
            pl.BlockSpec((1, acc_len), lambda: (0, 0)),
        ],
        out_specs=pl.BlockSpec((1, acc_len), lambda: (0, 0)),
    )(partials, btile)

    return out_flat.reshape(n_nodes, 2)


# speedup vs baseline: 31.9046x; 1.0352x over previous
import functools

import jax
import jax.numpy as jnp
from jax import lax
from jax.experimental import pallas as pl
from jax.experimental.pallas import tpu as pltpu
from jax.experimental.pallas import tpu_sc as plsc

_NC = 2
_NS = 16
_NW = _NC * _NS
_L = 16


def _feats_body(x_ref, w_in_ref, wcat_ref, c_ref, o_ref):
    m8 = lax.dot_general(w_in_ref[...], wcat_ref[...],
                         (((1,), (0,)), ((), ())),
                         precision=lax.Precision.HIGHEST,
                         preferred_element_type=jnp.float32)
    o_ref[...] = lax.dot_general(m8, x_ref[...],
                                 (((0,), (1,)), ((), ())),
                                 precision=lax.Precision.HIGHEST,
                                 preferred_element_type=jnp.float32) + c_ref[...]


def _reduce_body(p_ref, b_ref, o_ref):
    o_ref[...] = jnp.sum(p_ref[...], axis=0, keepdims=True) + b_ref[...]


def _make_sc_edge_kernel(n_nodes, n_edges):
    epw = n_edges // _NW
    acc_len = 2 * n_nodes
    mesh = plsc.VectorSubcoreMesh(core_axis_name="c", subcore_axis_name="s",
                                  num_cores=_NC, num_subcores=_NS)

    @functools.partial(
        pl.kernel,
        out_type=jax.ShapeDtypeStruct((_NW, acc_len), jnp.float32),
        mesh=mesh,
        compiler_params=pltpu.CompilerParams(needs_layout_passes=False),
        scratch_types=[
            pltpu.VMEM((epw,), jnp.int32),
            pltpu.VMEM((epw,), jnp.int32),
            pltpu.VMEM((n_nodes,), jnp.float32),
            pltpu.VMEM((n_nodes,), jnp.float32),
            pltpu.VMEM((n_nodes,), jnp.float32),
            pltpu.VMEM((n_nodes,), jnp.float32),
            pltpu.VMEM((acc_len,), jnp.float32),
        ],
    )
    def sc_edge_kernel(src_hbm, dst_hbm, feats_hbm, part_hbm,
                       srcv, dstv, g0v, g1v, pv, qv, accv):
        wid = lax.axis_index("s") * _NC + lax.axis_index("c")
        base = wid * epw
        pltpu.sync_copy(src_hbm.at[pl.ds(base, epw)], srcv)
        pltpu.sync_copy(dst_hbm.at[pl.ds(base, epw)], dstv)
        pltpu.sync_copy(feats_hbm.at[0], g0v)
        pltpu.sync_copy(feats_hbm.at[1], g1v)
        pltpu.sync_copy(feats_hbm.at[2], pv)
        pltpu.sync_copy(feats_hbm.at[3], qv)

        zunroll = 10
        zeros = jnp.zeros((_L,), jnp.float32)

        def zero_body(i, carry):
            for j in range(zunroll):
                accv[pl.ds((i * zunroll + j) * _L, _L)] = zeros
            return carry
        lax.fori_loop(0, acc_len // (_L * zunroll), zero_body, 0)

        unroll = 5

        def edge_body(i, carry):
            for j in range(unroll):
                off = (i * unroll + j) * _L
                s = srcv[pl.ds(off, _L)]
                d = dstv[pl.ds(off, _L)]
                pp = plsc.load_gather(pv, [s])
                qq = plsc.load_gather(qv, [d])
                w = 1.0 / (1.0 + jnp.exp(-(pp + qq)))
                w0 = w * plsc.load_gather(g0v, [s])
                w1 = w * plsc.load_gather(g1v, [s])
                d2 = d + d
                plsc.addupdate_scatter(accv, [d2], w0)
                plsc.addupdate_scatter(accv, [d2 + 1], w1)
            return carry
        lax.fori_loop(0, epw // (_L * unroll), edge_body, 0)

        pltpu.sync_copy(accv, part_hbm.at[wid])

    return sc_edge_kernel


def kernel(x, edge_index, W_in, b_in, W_edge, b_edge, W_out, b_out):
    n_nodes, in_feats = x.shape
    n_edges = edge_index.shape[1]
    hidden = W_in.shape[1]

    wcat = jnp.concatenate(
        [W_out, W_edge[:hidden], W_edge[hidden:],
         jnp.zeros((hidden, 4), jnp.float32)], axis=1)
    ext = jnp.zeros((8,), jnp.float32).at[3].set(b_edge[0])
    c_col = (b_in @ wcat + ext).reshape(8, 1)

    feats = pl.pallas_call(
        _feats_body,
        out_shape=jax.ShapeDtypeStruct((8, n_nodes), jnp.float32),
        in_specs=[
            pl.BlockSpec((n_nodes, in_feats), lambda: (0, 0)),
            pl.BlockSpec((in_feats, hidden), lambda: (0, 0)),
            pl.BlockSpec((hidden, 8), lambda: (0, 0)),
            pl.BlockSpec((8, 1), lambda: (0, 0)),
        ],
        out_specs=pl.BlockSpec((8, n_nodes), lambda: (0, 0)),
    )(x, W_in, wcat, c_col)

    src = edge_index[0]
    dst = edge_index[1]
    partials = _make_sc_edge_kernel(n_nodes, n_edges)(src, dst, feats)

    acc_len = 2 * n_nodes
    btile = jnp.tile(b_out, acc_len // 2).reshape(1, acc_len)
    out_flat = pl.pallas_call(
        _reduce_body,
        out_shape=jax.ShapeDtypeStruct((1, acc_len), jnp.float32),
        in_specs=[
            pl.BlockSpec((_NW, acc_len), lambda: (0, 0)),
            pl.BlockSpec((1, acc_len), lambda: (0, 0)),
        ],
        out_specs=pl.BlockSpec((1, acc_len), lambda: (0, 0)),
    )(partials, btile)

    return out_flat.reshape(n_nodes, 2)


# speedup vs baseline: 38.3398x; 1.2017x over previous
import functools

import jax
import jax.numpy as jnp
from jax import lax
from jax.experimental import pallas as pl
from jax.experimental.pallas import tpu as pltpu
from jax.experimental.pallas import tpu_sc as plsc

_NC = 2
_NS = 16
_NW = _NC * _NS
_L = 16


def _feats_body(x_ref, w_in_ref, wcat_ref, c_ref, o_ref):
    m8 = lax.dot_general(w_in_ref[...], wcat_ref[...],
                         (((1,), (0,)), ((), ())),
                         precision=lax.Precision.HIGHEST,
                         preferred_element_type=jnp.float32)
    o_ref[...] = lax.dot_general(m8, x_ref[...],
                                 (((0,), (1,)), ((), ())),
                                 precision=lax.Precision.HIGHEST,
                                 preferred_element_type=jnp.float32) + c_ref[...]


def _reduce_body(p_ref, b_ref, o_ref):
    o_ref[...] = jnp.sum(p_ref[...], axis=0, keepdims=True) + b_ref[...]


def _make_sc_edge_kernel(n_nodes, n_edges):
    epw = n_edges // _NW
    acc_len = 2 * n_nodes
    mesh = plsc.VectorSubcoreMesh(core_axis_name="c", subcore_axis_name="s",
                                  num_cores=_NC, num_subcores=_NS)

    @functools.partial(
        pl.kernel,
        out_type=jax.ShapeDtypeStruct((_NW, acc_len), jnp.float32),
        mesh=mesh,
        compiler_params=pltpu.CompilerParams(needs_layout_passes=False),
        scratch_types=[
            pltpu.VMEM((epw,), jnp.int32),
            pltpu.VMEM((epw,), jnp.int32),
            pltpu.VMEM((n_nodes,), jnp.float32),
            pltpu.VMEM((n_nodes,), jnp.float32),
            pltpu.VMEM((n_nodes,), jnp.float32),
            pltpu.VMEM((n_nodes,), jnp.float32),
            pltpu.VMEM((acc_len,), jnp.float32),
        ],
    )
    def sc_edge_kernel(src_hbm, dst_hbm, feats_hbm, part_hbm,
                       srcv, dstv, g0v, g1v, pv, qv, accv):
        wid = lax.axis_index("s") * _NC + lax.axis_index("c")
        base = wid * epw
        pltpu.sync_copy(src_hbm.at[pl.ds(base, epw)], srcv)
        pltpu.sync_copy(dst_hbm.at[pl.ds(base, epw)], dstv)
        pltpu.sync_copy(feats_hbm.at[0], g0v)
        pltpu.sync_copy(feats_hbm.at[1], g1v)
        pltpu.sync_copy(feats_hbm.at[2], pv)
        pltpu.sync_copy(feats_hbm.at[3], qv)

        zeros = jnp.zeros((_L,), jnp.float32)

        @plsc.parallel_loop(0, acc_len // _L, 1, unroll=10)
        def zero_body(i):
            accv[pl.ds(i * _L, _L)] = zeros

        @plsc.parallel_loop(0, epw // _L, 1, unroll=5)
        def edge_body(i):
            off = i * _L
            s = srcv[pl.ds(off, _L)]
            d = dstv[pl.ds(off, _L)]
            pp = plsc.load_gather(pv, [s])
            qq = plsc.load_gather(qv, [d])
            w = 1.0 / (1.0 + jnp.exp(-(pp + qq)))
            w0 = w * plsc.load_gather(g0v, [s])
            w1 = w * plsc.load_gather(g1v, [s])
            d2 = d + d
            plsc.addupdate_scatter(accv, [d2], w0)
            plsc.addupdate_scatter(accv, [d2 + 1], w1)

        pltpu.sync_copy(accv, part_hbm.at[wid])

    return sc_edge_kernel


def kernel(x, edge_index, W_in, b_in, W_edge, b_edge, W_out, b_out):
    n_nodes, in_feats = x.shape
    n_edges = edge_index.shape[1]
    hidden = W_in.shape[1]

    wcat = jnp.concatenate(
        [W_out, W_edge[:hidden], W_edge[hidden:],
         jnp.zeros((hidden, 4), jnp.float32)], axis=1)
    ext = jnp.zeros((8,), jnp.float32).at[3].set(b_edge[0])
    c_col = (b_in @ wcat + ext).reshape(8, 1)

    feats = pl.pallas_call(
        _feats_body,
        out_shape=jax.ShapeDtypeStruct((8, n_nodes), jnp.float32),
        in_specs=[
            pl.BlockSpec((n_nodes, in_feats), lambda: (0, 0)),
            pl.BlockSpec((in_feats, hidden), lambda: (0, 0)),
            pl.BlockSpec((hidden, 8), lambda: (0, 0)),
            pl.BlockSpec((8, 1), lambda: (0, 0)),
        ],
        out_specs=pl.BlockSpec((8, n_nodes), lambda: (0, 0)),
    )(x, W_in, wcat, c_col)

    src = edge_index[0]
    dst = edge_index[1]
    partials = _make_sc_edge_kernel(n_nodes, n_edges)(src, dst, feats)

    acc_len = 2 * n_nodes
    btile = jnp.tile(b_out, acc_len // 2).reshape(1, acc_len)
    out_flat = pl.pallas_call(
        _reduce_body,
        out_shape=jax.ShapeDtypeStruct((1, acc_len), jnp.float32),
        in_specs=[
            pl.BlockSpec((_NW, acc_len), lambda: (0, 0)),
            pl.BlockSpec((1, acc_len), lambda: (0, 0)),
        ],
        out_specs=pl.BlockSpec((1, acc_len), lambda: (0, 0)),
    )(partials, btile)

    return out_flat.reshape(n_nodes, 2)


# speedup vs baseline: 39.1536x; 1.0212x over previous
import functools

import jax
import jax.numpy as jnp
from jax import lax
from jax.experimental import pallas as pl
from jax.experimental.pallas import tpu as pltpu
from jax.experimental.pallas import tpu_sc as plsc

_NC = 2
_NS = 16
_NW = _NC * _NS
_L = 16


def _feats_body(x_ref, w_in_ref, wcat_ref, c_ref, o_ref):
    m8 = lax.dot_general(w_in_ref[...], wcat_ref[...],
                         (((1,), (0,)), ((), ())),
                         precision=lax.Precision.HIGHEST,
                         preferred_element_type=jnp.float32)
    o_ref[...] = lax.dot_general(m8, x_ref[...],
                                 (((0,), (1,)), ((), ())),
                                 preferred_element_type=jnp.float32) + c_ref[...]


def _reduce_body(p_ref, b_ref, o_ref):
    parity = lax.broadcasted_iota(jnp.int32, o_ref.shape, 1) & 1
    bias = jnp.where(parity == 0, b_ref[0], b_ref[1])
    o_ref[...] = jnp.sum(p_ref[...], axis=0, keepdims=True) + bias


def _make_sc_edge_kernel(n_nodes, n_edges):
    epw = n_edges // _NW
    acc_len = 2 * n_nodes
    mesh = plsc.VectorSubcoreMesh(core_axis_name="c", subcore_axis_name="s",
                                  num_cores=_NC, num_subcores=_NS)

    @functools.partial(
        pl.kernel,
        out_type=jax.ShapeDtypeStruct((_NW, acc_len), jnp.float32),
        mesh=mesh,
        compiler_params=pltpu.CompilerParams(needs_layout_passes=False),
        scratch_types=[
            pltpu.VMEM((epw,), jnp.int32),
            pltpu.VMEM((n_nodes,), jnp.float32),
            pltpu.VMEM((n_nodes,), jnp.float32),
            pltpu.VMEM((n_nodes,), jnp.float32),
            pltpu.VMEM((n_nodes,), jnp.float32),
            pltpu.VMEM((acc_len,), jnp.float32),
            pltpu.SemaphoreType.DMA,
        ],
    )
    def sc_edge_kernel(ep_hbm, feats_hbm, part_hbm,
                       epv, g0v, g1v, pv, qv, accv, sem):
        wid = lax.axis_index("s") * _NC + lax.axis_index("c")
        base = wid * epw
        cps = [
            pltpu.async_copy(ep_hbm.at[pl.ds(base, epw)], epv, sem),
            pltpu.async_copy(feats_hbm.at[0], g0v, sem),
            pltpu.async_copy(feats_hbm.at[1], g1v, sem),
            pltpu.async_copy(feats_hbm.at[2], pv, sem),
            pltpu.async_copy(feats_hbm.at[3], qv, sem),
        ]

        zeros = jnp.zeros((_L,), jnp.float32)

        @plsc.parallel_loop(0, acc_len // _L, 1, unroll=10)
        def zero_body(i):
            accv[pl.ds(i * _L, _L)] = zeros

        for cp in cps:
            cp.wait()

        @plsc.parallel_loop(0, epw // _L, 1, unroll=5)
        def edge_body(i):
            off = i * _L
            v = epv[pl.ds(off, _L)]
            s = v & 0xFFFF
            d = lax.shift_right_logical(v, 16)
            pp = plsc.load_gather(pv, [s])
            qq = plsc.load_gather(qv, [d])
            w = 1.0 / (1.0 + jnp.exp(-(pp + qq)))
            w0 = w * plsc.load_gather(g0v, [s])
            w1 = w * plsc.load_gather(g1v, [s])
            d2 = d + d
            plsc.addupdate_scatter(accv, [d2], w0)
            plsc.addupdate_scatter(accv, [d2 + 1], w1)

        pltpu.sync_copy(accv, part_hbm.at[wid])

    return sc_edge_kernel


def kernel(x, edge_index, W_in, b_in, W_edge, b_edge, W_out, b_out):
    n_nodes, in_feats = x.shape
    n_edges = edge_index.shape[1]
    hidden = W_in.shape[1]

    wcat = jnp.concatenate(
        [W_out, W_edge[:hidden], W_edge[hidden:],
         jnp.zeros((hidden, 4), jnp.float32)], axis=1)
    ext = jnp.zeros((8,), jnp.float32).at[3].set(b_edge[0])
    c_col = (b_in @ wcat + ext).reshape(8, 1)

    nblk = 1024
    grid = (n_nodes + nblk - 1) // nblk
    feats = pl.pallas_call(
        _feats_body,
        grid=(grid,),
        out_shape=jax.ShapeDtypeStruct((8, n_nodes), jnp.float32),
        in_specs=[
            pl.BlockSpec((nblk, in_feats), lambda i: (i, 0)),
            pl.BlockSpec((in_feats, hidden), lambda i: (0, 0)),
            pl.BlockSpec((hidden, 8), lambda i: (0, 0)),
            pl.BlockSpec((8, 1), lambda i: (0, 0)),
        ],
        out_specs=pl.BlockSpec((8, nblk), lambda i: (0, i)),
    )(x, W_in, wcat, c_col)

    epacked = (edge_index[1] << 16) | edge_index[0]
    partials = _make_sc_edge_kernel(n_nodes, n_edges)(epacked, feats)

    acc_len = 2 * n_nodes
    out_flat = pl.pallas_call(
        _reduce_body,
        out_shape=jax.ShapeDtypeStruct((1, acc_len), jnp.float32),
        in_specs=[
            pl.BlockSpec((_NW, acc_len), lambda: (0, 0)),
            pl.BlockSpec(memory_space=pltpu.SMEM),
        ],
        out_specs=pl.BlockSpec((1, acc_len), lambda: (0, 0)),
    )(partials, b_out)

    return out_flat.reshape(n_nodes, 2)


# speedup vs baseline: 41.9268x; 1.0708x over previous
import functools

import jax
import jax.numpy as jnp
from jax import lax
from jax.experimental import pallas as pl
from jax.experimental.pallas import tpu as pltpu
from jax.experimental.pallas import tpu_sc as plsc

_NC = 2
_NS = 16
_NW = _NC * _NS
_L = 16


def _feats_body(x_ref, w_in_ref, wcat_ref, c_ref, o_ref):
    m8 = lax.dot_general(w_in_ref[...], wcat_ref[...],
                         (((1,), (0,)), ((), ())),
                         precision=lax.Precision.HIGHEST,
                         preferred_element_type=jnp.float32)
    o_ref[...] = lax.dot_general(m8, x_ref[...],
                                 (((0,), (1,)), ((), ())),
                                 preferred_element_type=jnp.float32) + c_ref[...]


def _reduce_body(p_ref, b_ref, o_ref):
    parity = lax.broadcasted_iota(jnp.int32, o_ref.shape, 1) & 1
    bias = jnp.where(parity == 0, b_ref[0], b_ref[1])
    o_ref[...] = jnp.sum(p_ref[...], axis=0, keepdims=True) + bias


def _make_sc_edge_kernel(n_nodes, n_edges):
    epw = n_edges // _NW
    acc_len = 2 * n_nodes
    mesh = plsc.VectorSubcoreMesh(core_axis_name="c", subcore_axis_name="s",
                                  num_cores=_NC, num_subcores=_NS)

    @functools.partial(
        pl.kernel,
        out_type=jax.ShapeDtypeStruct((_NW, acc_len), jnp.float32),
        mesh=mesh,
        compiler_params=pltpu.CompilerParams(needs_layout_passes=False),
        scratch_types=[
            pltpu.VMEM((epw,), jnp.int32),
            pltpu.VMEM((n_nodes,), jnp.float32),
            pltpu.VMEM((n_nodes,), jnp.float32),
            pltpu.VMEM((n_nodes,), jnp.float32),
            pltpu.VMEM((n_nodes,), jnp.float32),
            pltpu.VMEM((acc_len,), jnp.float32),
            pltpu.SemaphoreType.DMA,
        ],
    )
    def sc_edge_kernel(ep_hbm, feats_hbm, part_hbm,
                       epv, g0v, g1v, pv, qv, accv, sem):
        wid = lax.axis_index("s") * _NC + lax.axis_index("c")
        base = wid * epw
        cps = [
            pltpu.async_copy(ep_hbm.at[pl.ds(base, epw)], epv, sem),
            pltpu.async_copy(feats_hbm.at[0], g0v, sem),
            pltpu.async_copy(feats_hbm.at[1], g1v, sem),
            pltpu.async_copy(feats_hbm.at[2], pv, sem),
            pltpu.async_copy(feats_hbm.at[3], qv, sem),
        ]

        zeros = jnp.zeros((_L,), jnp.float32)

        @plsc.parallel_loop(0, acc_len // _L, 1, unroll=10)
        def zero_body(i):
            accv[pl.ds(i * _L, _L)] = zeros

        for cp in cps:
            cp.wait()

        @plsc.parallel_loop(0, epw // _L, 1, unroll=5)
        def edge_body(i):
            off = i * _L
            v = epv[pl.ds(off, _L)]
            s = v & 0xFFFF
            d = lax.shift_right_logical(v, 16)
            pp = plsc.load_gather(pv, [s])
            qq = plsc.load_gather(qv, [d])
            w = 1.0 / (1.0 + jnp.exp(-(pp + qq)))
            w0 = w * plsc.load_gather(g0v, [s])
            w1 = w * plsc.load_gather(g1v, [s])
            d2 = d + d
            plsc.addupdate_scatter(accv, [d2], w0)
            plsc.addupdate_scatter(accv, [d2 + 1], w1)

        pltpu.sync_copy(accv, part_hbm.at[wid])

    return sc_edge_kernel


def kernel(x, edge_index, W_in, b_in, W_edge, b_edge, W_out, b_out):
    n_nodes, in_feats = x.shape
    n_edges = edge_index.shape[1]
    hidden = W_in.shape[1]

    wcat = jnp.concatenate(
        [W_out, W_edge[:hidden], W_edge[hidden:],
         jnp.zeros((hidden, 4), jnp.float32)], axis=1)
    ext = jnp.zeros((8,), jnp.float32).at[3].set(b_edge[0])
    c_col = (b_in @ wcat + ext).reshape(8, 1)

    feats = pl.pallas_call(
        _feats_body,
        out_shape=jax.ShapeDtypeStruct((8, n_nodes), jnp.float32),
        in_specs=[
            pl.BlockSpec((n_nodes, in_feats), lambda: (0, 0)),
            pl.BlockSpec((in_feats, hidden), lambda: (0, 0)),
            pl.BlockSpec((hidden, 8), lambda: (0, 0)),
            pl.BlockSpec((8, 1), lambda: (0, 0)),
        ],
        out_specs=pl.BlockSpec((8, n_nodes), lambda: (0, 0)),
    )(x, W_in, wcat, c_col)

    epacked = (edge_index[1] << 16) | edge_index[0]
    partials = _make_sc_edge_kernel(n_nodes, n_edges)(epacked, feats)

    acc_len = 2 * n_nodes
    out_flat = pl.pallas_call(
        _reduce_body,
        out_shape=jax.ShapeDtypeStruct((1, acc_len), jnp.float32),
        in_specs=[
            pl.BlockSpec((_NW, acc_len), lambda: (0, 0)),
            pl.BlockSpec(memory_space=pltpu.SMEM),
        ],
        out_specs=pl.BlockSpec((1, acc_len), lambda: (0, 0)),
    )(partials, b_out)

    return out_flat.reshape(n_nodes, 2)


# speedup vs baseline: 52.3904x; 1.2496x over previous
import functools

import jax
import jax.numpy as jnp
from jax import lax
from jax.experimental import pallas as pl
from jax.experimental.pallas import tpu as pltpu
from jax.experimental.pallas import tpu_sc as plsc

_NC = 2
_NS = 16
_NW = _NC * _NS
_L = 16


def _feats_body(x_ref, w_in_ref, wcat_ref, c_ref, o_ref):
    m8 = lax.dot_general(w_in_ref[...], wcat_ref[...],
                         (((1,), (0,)), ((), ())),
                         precision=lax.Precision.HIGHEST,
                         preferred_element_type=jnp.float32)
    o_ref[...] = lax.dot_general(m8, x_ref[...],
                                 (((0,), (1,)), ((), ())),
                                 preferred_element_type=jnp.float32) + c_ref[...]


def _reduce_body(p_ref, b_ref, o_ref):
    parity = lax.broadcasted_iota(jnp.int32, o_ref.shape, 1) & 1
    bias = jnp.where(parity == 0, b_ref[0], b_ref[1])
    o_ref[...] = jnp.sum(p_ref[...], axis=0, keepdims=True) + bias


def _make_sc_edge_kernel(n_nodes, n_edges):
    cols = n_edges // 128
    cpw = cols // _NW
    rem = cols - cpw * _NW
    epw = cpw * 128
    acc_len = 2 * n_nodes
    mesh = plsc.VectorSubcoreMesh(core_axis_name="c", subcore_axis_name="s",
                                  num_cores=_NC, num_subcores=_NS)

    @functools.partial(
        pl.kernel,
        out_type=jax.ShapeDtypeStruct((_NW, acc_len), jnp.float32),
        mesh=mesh,
        compiler_params=pltpu.CompilerParams(needs_layout_passes=False),
        scratch_types=[
            pltpu.VMEM((2, epw), jnp.int32),
            pltpu.VMEM((2, 128), jnp.int32),
            pltpu.VMEM((n_nodes,), jnp.float32),
            pltpu.VMEM((n_nodes,), jnp.float32),
            pltpu.VMEM((n_nodes,), jnp.float32),
            pltpu.VMEM((n_nodes,), jnp.float32),
            pltpu.VMEM((acc_len,), jnp.float32),
            pltpu.SemaphoreType.DMA,
        ],
    )
    def sc_edge_kernel(ei_hbm, feats_hbm, part_hbm,
                       epv, exv, g0v, g1v, pv, qv, accv, sem):
        wid = lax.axis_index("s") * _NC + lax.axis_index("c")
        base = wid * epw
        exbase = (cpw * _NW + jnp.minimum(wid, rem - 1)) * 128
        cps = [
            pltpu.async_copy(ei_hbm.at[:, pl.ds(base, epw)], epv, sem),
            pltpu.async_copy(ei_hbm.at[:, pl.ds(exbase, 128)], exv, sem),
            pltpu.async_copy(feats_hbm.at[0], g0v, sem),
            pltpu.async_copy(feats_hbm.at[1], g1v, sem),
            pltpu.async_copy(feats_hbm.at[2], pv, sem),
            pltpu.async_copy(feats_hbm.at[3], qv, sem),
        ]

        zeros = jnp.zeros((_L,), jnp.float32)

        @plsc.parallel_loop(0, acc_len // _L, 1, unroll=10)
        def zero_body(i):
            accv[pl.ds(i * _L, _L)] = zeros

        for cp in cps:
            cp.wait()

        def do_edges(s, d):
            pp = plsc.load_gather(pv, [s])
            qq = plsc.load_gather(qv, [d])
            w = 1.0 / (1.0 + jnp.exp(-(pp + qq)))
            w0 = w * plsc.load_gather(g0v, [s])
            w1 = w * plsc.load_gather(g1v, [s])
            d2 = d + d
            plsc.addupdate_scatter(accv, [d2], w0)
            plsc.addupdate_scatter(accv, [d2 + 1], w1)

        @plsc.parallel_loop(0, epw // _L, 1, unroll=4)
        def edge_body(i):
            off = i * _L
            do_edges(epv[0, pl.ds(off, _L)], epv[1, pl.ds(off, _L)])

        @pl.when(wid < rem)
        def extra_edges():
            @plsc.parallel_loop(0, 128 // _L, 1, unroll=4)
            def extra_body(i):
                off = i * _L
                do_edges(exv[0, pl.ds(off, _L)], exv[1, pl.ds(off, _L)])

        pltpu.sync_copy(accv, part_hbm.at[wid])

    return sc_edge_kernel


def kernel(x, edge_index, W_in, b_in, W_edge, b_edge, W_out, b_out):
    n_nodes, in_feats = x.shape
    n_edges = edge_index.shape[1]
    hidden = W_in.shape[1]

    wcat = jnp.concatenate(
        [W_out, W_edge[:hidden], W_edge[hidden:],
         jnp.zeros((hidden, 4), jnp.float32)], axis=1)
    ext = jnp.zeros((8,), jnp.float32).at[3].set(b_edge[0])
    c_col = (b_in @ wcat + ext).reshape(8, 1)

    feats = pl.pallas_call(
        _feats_body,
        out_shape=jax.ShapeDtypeStruct((8, n_nodes), jnp.float32),
        in_specs=[
            pl.BlockSpec((n_nodes, in_feats), lambda: (0, 0)),
            pl.BlockSpec((in_feats, hidden), lambda: (0, 0)),
            pl.BlockSpec((hidden, 8), lambda: (0, 0)),
            pl.BlockSpec((8, 1), lambda: (0, 0)),
        ],
        out_specs=pl.BlockSpec((8, n_nodes), lambda: (0, 0)),
    )(x, W_in, wcat, c_col)

    partials = _make_sc_edge_kernel(n_nodes, n_edges)(edge_index, feats)

    acc_len = 2 * n_nodes
    out_flat = pl.pallas_call(
        _reduce_body,
        out_shape=jax.ShapeDtypeStruct((1, acc_len), jnp.float32),
        in_specs=[
            pl.BlockSpec((_NW, acc_len), lambda: (0, 0)),
            pl.BlockSpec(memory_space=pltpu.SMEM),
        ],
        out_specs=pl.BlockSpec((1, acc_len), lambda: (0, 0)),
    )(partials, b_out)

    return out_flat.reshape(n_nodes, 2)


# speedup vs baseline: 52.4798x; 1.0017x over previous
import functools

import jax
import jax.numpy as jnp
from jax import lax
from jax.experimental import pallas as pl
from jax.experimental.pallas import tpu as pltpu
from jax.experimental.pallas import tpu_sc as plsc

_NC = 2
_NS = 16
_NW = _NC * _NS
_L = 16


def _feats_body(x_ref, w_in_ref, wcat_ref, c_ref, o_ref):
    m8 = lax.dot_general(w_in_ref[...], wcat_ref[...],
                         (((1,), (0,)), ((), ())),
                         precision=lax.Precision.HIGHEST,
                         preferred_element_type=jnp.float32)
    o_ref[...] = lax.dot_general(m8, x_ref[...],
                                 (((0,), (1,)), ((), ())),
                                 preferred_element_type=jnp.float32) + c_ref[...]


def _reduce_body(p_ref, b_ref, o_ref):
    parity = lax.broadcasted_iota(jnp.int32, o_ref.shape, 1) & 1
    bias = jnp.where(parity == 0, b_ref[0], b_ref[1])
    o_ref[...] = jnp.sum(p_ref[...], axis=0, keepdims=True) + bias


def _make_sc_edge_kernel(n_nodes, n_edges):
    cols = n_edges // 128
    cpw = cols // _NW
    rem = cols - cpw * _NW
    epw = cpw * 128
    acc_len = 2 * n_nodes
    mesh = plsc.VectorSubcoreMesh(core_axis_name="c", subcore_axis_name="s",
                                  num_cores=_NC, num_subcores=_NS)

    @functools.partial(
        pl.kernel,
        out_type=jax.ShapeDtypeStruct((_NW, acc_len), jnp.float32),
        mesh=mesh,
        compiler_params=pltpu.CompilerParams(needs_layout_passes=False),
        scratch_types=[
            pltpu.VMEM((2, epw), jnp.int32),
            pltpu.VMEM((2, 128), jnp.int32),
            pltpu.VMEM((n_nodes,), jnp.float32),
            pltpu.VMEM((n_nodes,), jnp.float32),
            pltpu.VMEM((n_nodes,), jnp.float32),
            pltpu.VMEM((n_nodes,), jnp.float32),
            pltpu.VMEM((acc_len,), jnp.float32),
            pltpu.SemaphoreType.DMA,
        ],
    )
    def sc_edge_kernel(ei_hbm, feats_hbm, part_hbm,
                       epv, exv, g0v, g1v, pv, qv, accv, sem):
        wid = lax.axis_index("s") * _NC + lax.axis_index("c")
        base = wid * epw
        exbase = (cpw * _NW + jnp.minimum(wid, rem - 1)) * 128
        cps = [
            pltpu.async_copy(ei_hbm.at[:, pl.ds(base, epw)], epv, sem),
            pltpu.async_copy(ei_hbm.at[:, pl.ds(exbase, 128)], exv, sem),
            pltpu.async_copy(feats_hbm.at[0], g0v, sem),
            pltpu.async_copy(feats_hbm.at[1], g1v, sem),
            pltpu.async_copy(feats_hbm.at[2], pv, sem),
            pltpu.async_copy(feats_hbm.at[3], qv, sem),
        ]

        zeros = jnp.zeros((_L,), jnp.float32)

        @plsc.parallel_loop(0, acc_len // _L, 1, unroll=10)
        def zero_body(i):
            accv[pl.ds(i * _L, _L)] = zeros

        for cp in cps:
            cp.wait()

        def do_edges(s, d):
            pp = plsc.load_gather(pv, [s])
            qq = plsc.load_gather(qv, [d])
            w = 1.0 / (1.0 + jnp.exp(-(pp + qq)))
            w0 = w * plsc.load_gather(g0v, [s])
            w1 = w * plsc.load_gather(g1v, [s])
            d2 = d + d
            plsc.addupdate_scatter(accv, [d2], w0)
            plsc.addupdate_scatter(accv, [d2 + 1], w1)

        @plsc.parallel_loop(0, epw // _L, 1, unroll=8)
        def edge_body(i):
            off = i * _L
            do_edges(epv[0, pl.ds(off, _L)], epv[1, pl.ds(off, _L)])

        @pl.when(wid < rem)
        def extra_edges():
            @plsc.parallel_loop(0, 128 // _L, 1, unroll=4)
            def extra_body(i):
                off = i * _L
                do_edges(exv[0, pl.ds(off, _L)], exv[1, pl.ds(off, _L)])

        pltpu.sync_copy(accv, part_hbm.at[wid])

    return sc_edge_kernel


def kernel(x, edge_index, W_in, b_in, W_edge, b_edge, W_out, b_out):
    n_nodes, in_feats = x.shape
    n_edges = edge_index.shape[1]
    hidden = W_in.shape[1]

    wcat = jnp.concatenate(
        [W_out, W_edge[:hidden], W_edge[hidden:],
         jnp.zeros((hidden, 4), jnp.float32)], axis=1)
    ext = jnp.zeros((8,), jnp.float32).at[3].set(b_edge[0])
    c_col = (b_in @ wcat + ext).reshape(8, 1)

    feats = pl.pallas_call(
        _feats_body,
        out_shape=jax.ShapeDtypeStruct((8, n_nodes), jnp.float32),
        in_specs=[
            pl.BlockSpec((n_nodes, in_feats), lambda: (0, 0)),
            pl.BlockSpec((in_feats, hidden), lambda: (0, 0)),
            pl.BlockSpec((hidden, 8), lambda: (0, 0)),
            pl.BlockSpec((8, 1), lambda: (0, 0)),
        ],
        out_specs=pl.BlockSpec((8, n_nodes), lambda: (0, 0)),
    )(x, W_in, wcat, c_col)

    partials = _make_sc_edge_kernel(n_nodes, n_edges)(edge_index, feats)

    acc_len = 2 * n_nodes
    out_flat = pl.pallas_call(
        _reduce_body,
        out_shape=jax.ShapeDtypeStruct((1, acc_len), jnp.float32),
        in_specs=[
            pl.BlockSpec((_NW, acc_len), lambda: (0, 0)),
            pl.BlockSpec(memory_space=pltpu.SMEM),
        ],
        out_specs=pl.BlockSpec((1, acc_len), lambda: (0, 0)),
    )(partials, b_out)

    return out_flat.reshape(n_nodes, 2)
